# baseline (speedup 1.0000x reference)
import functools

import jax
import jax.numpy as jnp
from jax.experimental import pallas as pl
from jax.experimental.pallas import tpu as pltpu

_LOW_BOUND = 1
_LOG2E = 1.4426950408889634

_TR = 2048
_SUB = 512


def _stream_body(n, total_rows, x_hbm, w1_ref, b1_ref, w2_ref, b2_ref,
                 dec_ref, keep_ref, xbuf, sems):
    num_tiles = total_rows // _TR

    def start_copy(i, slot):
        pltpu.make_async_copy(
            x_hbm.at[pl.ds(i * _TR, _TR), :], xbuf.at[slot], sems.at[slot]
        ).start()

    def wait_copy(i, slot):
        pltpu.make_async_copy(
            x_hbm.at[pl.ds(i * _TR, _TR), :], xbuf.at[slot], sems.at[slot]
        ).wait()

    start_copy(0, 0)
    for i in range(num_tiles):
        slot = i % 2
        if i + 1 < num_tiles:
            start_copy(i + 1, (i + 1) % 2)
        wait_copy(i, slot)
        out_base = i * (_TR // n)
        for k in range(_TR // _SUB):
            xs = xbuf[slot, k * _SUB:(k + 1) * _SUB, :]
            h = jnp.dot(xs, w1_ref[...], preferred_element_type=jnp.float32)
            h = jnp.maximum(h + b1_ref[...], 0.0)
            logits = jnp.dot(h, w2_ref[...], preferred_element_type=jnp.float32)
            logits = logits + b2_ref[...]
            d = logits[:, 1:2] - logits[:, 0:1]
            rows = _SUB // n
            d = d.reshape(rows, n)
            dec = (d > 0.0).astype(jnp.float32)
            any_active = jnp.max(dec, axis=1, keepdims=True)
            col0 = jax.lax.broadcasted_iota(jnp.int32, dec.shape, 1) == 0
            dec = jnp.where((any_active == 0.0) & col0, 1.0, dec)
            lo = out_base + k * rows
            dec_ref[lo:lo + rows, :] = dec
            e = jnp.exp2(d * -_LOG2E)
            keep_ref[lo:lo + rows, :] = 1.0 / (1.0 + e)


@jax.jit
def kernel(slots, W1, b1, W2, b2, global_step):
    B, N, DIM = slots.shape
    F = W1.shape[1]
    x = slots.reshape(B * N, DIM)
    b1r = b1.reshape(1, F)
    b2r = b2.reshape(1, 2)

    out = pl.pallas_call(
        functools.partial(_stream_body, N, B * N),
        in_specs=[
            pl.BlockSpec(memory_space=pl.ANY),
            pl.BlockSpec((DIM, F), lambda: (0, 0)),
            pl.BlockSpec((1, F), lambda: (0, 0)),
            pl.BlockSpec((F, 2), lambda: (0, 0)),
            pl.BlockSpec((1, 2), lambda: (0, 0)),
        ],
        out_specs=[
            pl.BlockSpec((B, N), lambda: (0, 0)),
            pl.BlockSpec((B, N), lambda: (0, 0)),
        ],
        out_shape=[
            jax.ShapeDtypeStruct((B, N), jnp.float32),
            jax.ShapeDtypeStruct((B, N), jnp.float32),
        ],
        scratch_shapes=[
            pltpu.VMEM((2, _TR, DIM), jnp.float32),
            pltpu.SemaphoreType.DMA((2,)),
        ],
    )(x, W1, b1r, W2, b2r)
    return (out[0], out[1])
